# Initial kernel scaffold; baseline (speedup 1.0000x reference)
import jax
import jax.numpy as jnp
from jax.experimental import pallas as pl


def kernel(anom_scores, norm_scores):
    raise NotImplementedError("write your pallas kernel here")


# speedup vs baseline: 7.4085x; 7.4085x over previous
import jax
import jax.numpy as jnp
from jax.experimental import pallas as pl
from jax.experimental.pallas import tpu as pltpu

_TOPK = 3
_MARGIN = 100.0
_LAMBDA_SPARSITY = 0.008
_LAMBDA_SMOOTH = 0.0008
_HARD_NEG_RATIO = 0.3

_I32_MIN = -2147483648
_I32_MAGN = 2147483647


def _unkey(s):
    i = s ^ (jax.lax.shift_right_arithmetic(s, 31) & jnp.int32(_I32_MAGN))
    return jax.lax.bitcast_convert_type(i, jnp.float32)


def _topk_mean(x, k):
    rows = x.shape[0]
    kf = jnp.float32(k)

    def body(it, p):
        j = 31 - it
        bit = jax.lax.shift_left(jnp.int32(1), j)
        c = p | bit
        tf = _unkey(c ^ jnp.int32(_I32_MIN))
        cnt = jnp.sum(jnp.where(x >= tf, 1, 0).astype(jnp.int32),
                      axis=1, keepdims=True)
        return jnp.where(cnt >= k, c, p)

    p = jax.lax.fori_loop(0, 32, body, jnp.zeros((rows, 1), jnp.int32))
    tf = _unkey(p ^ jnp.int32(_I32_MIN))
    gt = x > tf
    n_gt = jnp.sum(gt.astype(jnp.float32), axis=1, keepdims=True)
    sum_gt = jnp.sum(jnp.where(gt, x, 0.0), axis=1, keepdims=True)
    return (sum_gt + (kf - n_gt) * tf) / kf


def _loss_kernel(anom_ref, norm_ref, total_ref, rank_ref, sp_ref, sm_ref):
    anom = anom_ref[...]
    norm = norm_ref[...]
    b_a, t_a = anom.shape
    b_n, t_n = norm.shape
    hard_k = max(1, int(t_n * _HARD_NEG_RATIO))

    anom_mean = _topk_mean(anom, min(_TOPK, t_a))
    norm_mean = _topk_mean(norm, hard_k)

    ones_col = jnp.ones((b_a, 1), jnp.float32)
    norm_row = jax.lax.dot_general(
        ones_col, norm_mean,
        dimension_numbers=(((1,), (1,)), ((), ())),
        preferred_element_type=jnp.float32,
    )
    pairs = jnp.maximum(_MARGIN - anom_mean + norm_row, 0.0)
    rank_loss = jnp.sum(pairs) / jnp.float32(b_a * b_n)

    sum_a = jnp.sum(anom)
    sum_n = jnp.sum(norm)
    sparsity = (sum_a / jnp.float32(b_a * t_a)
                + sum_n / jnp.float32(b_n * t_n)) * 0.5

    diff_a = anom[:, 1:] - anom[:, :-1]
    diff_n = norm[:, 1:] - norm[:, :-1]
    smooth = (jnp.sum(diff_a * diff_a) / jnp.float32(b_a * (t_a - 1))
              + jnp.sum(diff_n * diff_n) / jnp.float32(b_n * (t_n - 1))) * 0.5

    total_ref[0, 0] = rank_loss + _LAMBDA_SPARSITY * sparsity \
        + _LAMBDA_SMOOTH * smooth
    rank_ref[0, 0] = rank_loss
    sp_ref[0, 0] = sparsity
    sm_ref[0, 0] = smooth


def kernel(anom_scores, norm_scores):
    scalar = jax.ShapeDtypeStruct((1, 1), jnp.float32)
    smem = pl.BlockSpec(memory_space=pltpu.SMEM)
    total, rank, sp, sm = pl.pallas_call(
        _loss_kernel,
        out_shape=(scalar, scalar, scalar, scalar),
        in_specs=[pl.BlockSpec(memory_space=pltpu.VMEM)] * 2,
        out_specs=(smem, smem, smem, smem),
    )(anom_scores, norm_scores)
    return (total[0, 0], rank[0, 0], sp[0, 0], sm[0, 0])
